# initial kernel scaffold (unmeasured)
import jax
import jax.numpy as jnp
from jax import lax
from jax.experimental import pallas as pl
from jax.experimental.pallas import tpu as pltpu


def kernel(
    x,
):
    def body(*refs):
        pass

    out_shape = jax.ShapeDtypeStruct(..., jnp.float32)
    return pl.pallas_call(body, out_shape=out_shape)(...)


# baseline (device time: 424405 ns/iter reference)
import jax
import jax.numpy as jnp
from jax import lax
from jax.experimental import pallas as pl
from jax.experimental.pallas import tpu as pltpu

M, N = 16384, 2048
MH, NH = M // 2, N // 2


def kernel(x):
    my = lax.axis_index("y")
    xb = lax.dynamic_slice(x[0], (my * MH, 0), (MH, N)).astype(jnp.bfloat16)

    def body(x_ref, out_ref, local_ref, recvx_ref, result_ref,
             copy_sem, store_sem, sx_send, sx_recv, sy_send, sy_recv):
        mx = lax.axis_index("x")
        my = lax.axis_index("y")
        rows0 = my * MH
        mcol = mx * NH
        pcol = (1 - mx) * NH

        barrier_sem = pltpu.get_barrier_semaphore()
        for nbr in ((1 - mx, my), (mx, 1 - my)):
            pl.semaphore_signal(
                barrier_sem, inc=1,
                device_id=nbr, device_id_type=pl.DeviceIdType.MESH,
            )
        pl.semaphore_wait(barrier_sem, 2)

        cp = pltpu.make_async_copy(
            x_ref.at[:, pl.ds(mcol, NH)], local_ref, copy_sem)
        cp.start()

        rdma_x = pltpu.make_async_remote_copy(
            src_ref=x_ref.at[:, pl.ds(pcol, NH)],
            dst_ref=recvx_ref,
            send_sem=sx_send, recv_sem=sx_recv,
            device_id=(1 - mx, my), device_id_type=pl.DeviceIdType.MESH,
        )
        rdma_x.start()
        cp.wait()
        rdma_x.wait()

        result_ref[...] = local_ref[...] + recvx_ref[...]

        store = pltpu.make_async_copy(
            result_ref, out_ref.at[pl.ds(rows0, MH), :], store_sem)
        store.start()

        rdma_y = pltpu.make_async_remote_copy(
            src_ref=result_ref,
            dst_ref=out_ref.at[pl.ds(rows0, MH), :],
            send_sem=sy_send, recv_sem=sy_recv,
            device_id=(mx, 1 - my), device_id_type=pl.DeviceIdType.MESH,
        )
        rdma_y.start()
        store.wait()
        rdma_y.wait()

    return pl.pallas_call(
        body,
        out_shape=jax.ShapeDtypeStruct((M, NH), jnp.bfloat16),
        in_specs=[pl.BlockSpec(memory_space=pl.ANY)],
        out_specs=pl.BlockSpec(memory_space=pl.ANY),
        scratch_shapes=[
            pltpu.VMEM((MH, NH), jnp.bfloat16),
            pltpu.VMEM((MH, NH), jnp.bfloat16),
            pltpu.VMEM((MH, NH), jnp.bfloat16),
            pltpu.SemaphoreType.DMA,
            pltpu.SemaphoreType.DMA,
            pltpu.SemaphoreType.DMA,
            pltpu.SemaphoreType.DMA,
            pltpu.SemaphoreType.DMA,
            pltpu.SemaphoreType.DMA,
        ],
        compiler_params=pltpu.CompilerParams(
            collective_id=0, vmem_limit_bytes=63 * 1024 * 1024),
    )(xb)


# device time: 255295 ns/iter; 1.6624x vs baseline; 1.6624x over previous
import jax
import jax.numpy as jnp
from jax import lax
from jax.experimental import pallas as pl
from jax.experimental.pallas import tpu as pltpu

M, N = 16384, 2048
MH, NH = M // 2, N // 2
C = 16
CH = MH // C


def kernel(x):
    my = lax.axis_index("y")
    xb = lax.dynamic_slice(x[0], (my * MH, 0), (MH, N)).astype(jnp.bfloat16)

    def body(x_ref, out_ref, local_ref, recvx_ref, result_ref,
             copy_sem, store_sems, sx_send, sx_recv, sy_send, sy_recv):
        mx = lax.axis_index("x")
        my = lax.axis_index("y")
        rows0 = my * MH
        mcol = mx * NH
        pcol = (1 - mx) * NH

        barrier_sem = pltpu.get_barrier_semaphore()
        for nbr in ((1 - mx, my), (mx, 1 - my)):
            pl.semaphore_signal(
                barrier_sem, inc=1,
                device_id=nbr, device_id_type=pl.DeviceIdType.MESH,
            )
        pl.semaphore_wait(barrier_sem, 2)

        cp = pltpu.make_async_copy(
            x_ref.at[:, pl.ds(mcol, NH)], local_ref, copy_sem)
        cp.start()

        rdmas_x = []
        for c in range(C):
            r = pltpu.make_async_remote_copy(
                src_ref=x_ref.at[pl.ds(c * CH, CH), pl.ds(pcol, NH)],
                dst_ref=recvx_ref.at[pl.ds(c * CH, CH), :],
                send_sem=sx_send.at[c], recv_sem=sx_recv.at[c],
                device_id=(1 - mx, my), device_id_type=pl.DeviceIdType.MESH,
            )
            r.start()
            rdmas_x.append(r)

        cp.wait()

        stores = []
        rdmas_y = []
        for c in range(C):
            rdmas_x[c].wait_recv()
            result_ref[pl.ds(c * CH, CH), :] = (
                local_ref[pl.ds(c * CH, CH), :]
                + recvx_ref[pl.ds(c * CH, CH), :]
            )
            st = pltpu.make_async_copy(
                result_ref.at[pl.ds(c * CH, CH), :],
                out_ref.at[pl.ds(rows0 + c * CH, CH), :],
                store_sems.at[c],
            )
            st.start()
            stores.append(st)
            ry = pltpu.make_async_remote_copy(
                src_ref=result_ref.at[pl.ds(c * CH, CH), :],
                dst_ref=out_ref.at[pl.ds(rows0 + c * CH, CH), :],
                send_sem=sy_send.at[c], recv_sem=sy_recv.at[c],
                device_id=(mx, 1 - my), device_id_type=pl.DeviceIdType.MESH,
            )
            ry.start()
            rdmas_y.append(ry)

        for c in range(C):
            rdmas_x[c].wait_send()
            stores[c].wait()
            rdmas_y[c].wait()

    return pl.pallas_call(
        body,
        out_shape=jax.ShapeDtypeStruct((M, NH), jnp.bfloat16),
        in_specs=[pl.BlockSpec(memory_space=pl.ANY)],
        out_specs=pl.BlockSpec(memory_space=pl.ANY),
        scratch_shapes=[
            pltpu.VMEM((MH, NH), jnp.bfloat16),
            pltpu.VMEM((MH, NH), jnp.bfloat16),
            pltpu.VMEM((MH, NH), jnp.bfloat16),
            pltpu.SemaphoreType.DMA,
            pltpu.SemaphoreType.DMA((C,)),
            pltpu.SemaphoreType.DMA((C,)),
            pltpu.SemaphoreType.DMA((C,)),
            pltpu.SemaphoreType.DMA((C,)),
            pltpu.SemaphoreType.DMA((C,)),
        ],
        compiler_params=pltpu.CompilerParams(
            collective_id=0, vmem_limit_bytes=63 * 1024 * 1024),
    )(xb)


# device time: 222422 ns/iter; 1.9081x vs baseline; 1.1478x over previous
import jax
import jax.numpy as jnp
from jax import lax
from jax.experimental import pallas as pl
from jax.experimental.pallas import tpu as pltpu

M, N = 16384, 2048
MH, NH = M // 2, N // 2
C = 16
CH = MH // C


def kernel(x):
    def body(x_ref, out_ref, send_ref, recvx_ref, stage_p, stage_m,
             lp_sems, lm_sems, store_sems, sx_send, sx_recv, sy_send, sy_recv):
        mx = lax.axis_index("x")
        my = lax.axis_index("y")
        rows0 = my * MH
        mcol = mx * NH
        pcol = (1 - mx) * NH

        barrier_sem = pltpu.get_barrier_semaphore()
        for nbr in ((1 - mx, my), (mx, 1 - my)):
            pl.semaphore_signal(
                barrier_sem, inc=1,
                device_id=nbr, device_id_type=pl.DeviceIdType.MESH,
            )
        pl.semaphore_wait(barrier_sem, 2)

        def load(c, stage, sems, col0):
            return pltpu.make_async_copy(
                x_ref.at[0, pl.ds(rows0 + c * CH, CH), pl.ds(col0, NH)],
                stage.at[c % 2], sems.at[c % 2])

        loads_p = [load(c, stage_p, lp_sems, pcol) for c in range(C)]
        rdmas_x = []
        loads_p[0].start()
        for c in range(C):
            if c + 1 < C:
                loads_p[c + 1].start()
            loads_p[c].wait()
            send_ref[pl.ds(c * CH, CH), :] = (
                stage_p[c % 2].astype(jnp.bfloat16))
            r = pltpu.make_async_remote_copy(
                src_ref=send_ref.at[pl.ds(c * CH, CH), :],
                dst_ref=recvx_ref.at[pl.ds(c * CH, CH), :],
                send_sem=sx_send.at[c], recv_sem=sx_recv.at[c],
                device_id=(1 - mx, my), device_id_type=pl.DeviceIdType.MESH,
            )
            r.start()
            rdmas_x.append(r)

        loads_m = [load(c, stage_m, lm_sems, mcol) for c in range(C)]
        stores = []
        rdmas_y = []
        loads_m[0].start()
        for c in range(C):
            if c + 1 < C:
                loads_m[c + 1].start()
            loads_m[c].wait()
            rdmas_x[c].wait_recv()
            recvx_ref[pl.ds(c * CH, CH), :] = (
                stage_m[c % 2].astype(jnp.bfloat16)
                + recvx_ref[pl.ds(c * CH, CH), :]
            )
            st = pltpu.make_async_copy(
                recvx_ref.at[pl.ds(c * CH, CH), :],
                out_ref.at[pl.ds(rows0 + c * CH, CH), :],
                store_sems.at[c],
            )
            st.start()
            stores.append(st)
            ry = pltpu.make_async_remote_copy(
                src_ref=recvx_ref.at[pl.ds(c * CH, CH), :],
                dst_ref=out_ref.at[pl.ds(rows0 + c * CH, CH), :],
                send_sem=sy_send.at[c], recv_sem=sy_recv.at[c],
                device_id=(mx, 1 - my), device_id_type=pl.DeviceIdType.MESH,
            )
            ry.start()
            rdmas_y.append(ry)

        for c in range(C):
            rdmas_x[c].wait_send()
            stores[c].wait()
            rdmas_y[c].wait()

    return pl.pallas_call(
        body,
        out_shape=jax.ShapeDtypeStruct((M, NH), jnp.bfloat16),
        in_specs=[pl.BlockSpec(memory_space=pl.ANY)],
        out_specs=pl.BlockSpec(memory_space=pl.ANY),
        scratch_shapes=[
            pltpu.VMEM((MH, NH), jnp.bfloat16),
            pltpu.VMEM((MH, NH), jnp.bfloat16),
            pltpu.VMEM((2, CH, NH), jnp.float32),
            pltpu.VMEM((2, CH, NH), jnp.float32),
            pltpu.SemaphoreType.DMA((2,)),
            pltpu.SemaphoreType.DMA((2,)),
            pltpu.SemaphoreType.DMA((C,)),
            pltpu.SemaphoreType.DMA((C,)),
            pltpu.SemaphoreType.DMA((C,)),
            pltpu.SemaphoreType.DMA((C,)),
            pltpu.SemaphoreType.DMA((C,)),
        ],
        compiler_params=pltpu.CompilerParams(
            collective_id=0, vmem_limit_bytes=63 * 1024 * 1024),
    )(x)
